# Initial kernel scaffold; baseline (speedup 1.0000x reference)
import jax
import jax.numpy as jnp
from jax.experimental import pallas as pl


def kernel(feat_p3, feat_p4, feat_p5, feat_p6, feat_p7, cls_params, box_params, image_sizes):
    raise NotImplementedError("write your pallas kernel here")


# speedup vs baseline: 54.5256x; 54.5256x over previous
import functools

import jax
import jax.numpy as jnp
from jax.experimental import pallas as pl
from jax.experimental.pallas import tpu as pltpu

_CLASS_NUM = 80
_NMS_PRE = 1000
_LEVELS = ((64, 64, 8.0, 16.0),
           (32, 32, 16.0, 32.0),
           (16, 16, 32.0, 64.0),
           (8, 8, 64.0, 128.0),
           (4, 4, 128.0, 256.0))
_IOU_THR = 0.5


def _conv2d(x, w, b):
    y = jax.lax.conv_general_dilated(
        x, w, window_strides=(1, 1), padding='SAME',
        dimension_numbers=('NCHW', 'OIHW', 'NCHW'))
    return y + b[None, :, None, None]


def _head(x, params):
    for w, b in params[:-1]:
        x = jax.nn.relu(_conv2d(x, w, b))
    w, b = params[-1]
    return _conv2d(x, w, b)


def _decode_body(cls_ref, box_ref, out_ref, *, H, W, s, r):
    HW = H * W
    c = cls_ref[...]
    b = box_ref[...]
    smax = jnp.max(c, axis=1, keepdims=True)
    cid = jax.lax.broadcasted_iota(jnp.int32, c.shape, 1)
    am = jnp.min(jnp.where(c == smax, cid, _CLASS_NUM + 7), axis=1,
                 keepdims=True)
    p = jax.lax.broadcasted_iota(jnp.int32, (HW, 1), 0)
    xg = (p % W).astype(jnp.float32)
    yg = ((p // W) % H).astype(jnp.float32)
    cx = s * (xg + 0.5)
    cy = s * (yg + 0.5)
    x1 = cx - r * jnp.exp(b[:, 0:1])
    y1 = cy - r * jnp.exp(b[:, 1:2])
    x2 = cx + r * jnp.exp(b[:, 2:3])
    y2 = cy + r * jnp.exp(b[:, 3:4])
    lab = am.astype(jnp.float32)
    out_ref[...] = jnp.concatenate(
        (smax, lab, x1, y1, x2, y2, jnp.zeros((HW, 2), jnp.float32)), axis=1)


def _run_decode(cls_hw, box_hw, H, W, s, r):
    return pl.pallas_call(
        functools.partial(_decode_body, H=H, W=W, s=s, r=r),
        out_shape=jax.ShapeDtypeStruct((H * W, 8), jnp.float32),
    )(cls_hw, box_hw)


def _topk_body(cand_ref, vt_ref, out_ref, *, HW, K):
    s_row = vt_ref[0:1, :]
    jj = jax.lax.broadcasted_iota(jnp.int32, (1, HW), 1)
    CH = 512
    rank = jnp.zeros((1, HW), jnp.float32)
    for r0 in range(0, HW, CH):
        n = min(CH, HW - r0)
        sc = cand_ref[r0:r0 + n, 0:1]
        ii = jax.lax.broadcasted_iota(jnp.int32, (n, 1), 0) + r0
        win = (sc > s_row) | ((sc == s_row) & (ii < jj))
        rank = rank + jnp.sum(jnp.where(win, 1.0, 0.0), axis=0, keepdims=True)
    slot = jax.lax.broadcasted_iota(jnp.int32, (K, 1), 0).astype(jnp.float32)
    outs = [jnp.zeros((K, 1), jnp.float32) for _ in range(6)]
    for c0 in range(0, HW, CH):
        ohc = jnp.where(rank[:, c0:c0 + CH] == slot, 1.0, 0.0)
        for ch in range(6):
            vr = vt_ref[ch:ch + 1, c0:c0 + CH]
            outs[ch] = outs[ch] + jnp.sum(ohc * vr, axis=1, keepdims=True)
    out_ref[...] = jnp.concatenate(
        tuple(outs) + (jnp.zeros((K, 2), jnp.float32),), axis=1)


def _run_topk(cand, cand_t, HW, K):
    return pl.pallas_call(
        functools.partial(_topk_body, HW=HW, K=K),
        out_shape=jax.ShapeDtypeStruct((K, 8), jnp.float32),
    )(cand, cand_t)


def _nms_mat_body(cand_ref, ct_ref, mx_ref, m_ref, *, N, CH):
    i = pl.program_id(0)
    max_coord = mx_ref[0, 0]
    sc = cand_ref[:, 0:1]
    lab = cand_ref[:, 1:2]
    off = lab * (max_coord + 1.0)
    x1 = cand_ref[:, 2:3] + off
    y1 = cand_ref[:, 3:4] + off
    x2 = cand_ref[:, 4:5] + off
    y2 = cand_ref[:, 5:6] + off
    area = (x2 - x1) * (y2 - y1)

    lab_r = ct_ref[1:2, :]
    offr = lab_r * (max_coord + 1.0)
    x1r = ct_ref[2:3, :] + offr
    y1r = ct_ref[3:4, :] + offr
    x2r = ct_ref[4:5, :] + offr
    y2r = ct_ref[5:6, :] + offr
    arear = (x2r - x1r) * (y2r - y1r)
    srk = ct_ref[0:1, :]
    ii = jax.lax.broadcasted_iota(jnp.int32, (N, 1), 0)
    kk = jax.lax.broadcasted_iota(jnp.int32, (1, CH), 1) + i * CH

    X1 = jnp.maximum(x1, x1r)
    Y1 = jnp.maximum(y1, y1r)
    X2 = jnp.minimum(x2, x2r)
    Y2 = jnp.minimum(y2, y2r)
    inter = jnp.maximum(X2 - X1, 0.0) * jnp.maximum(Y2 - Y1, 0.0)
    union = area + arear - inter + 1e-6
    prec = (sc > srk) | ((sc == srk) & (ii < kk))
    m_ref[...] = jnp.where(prec & (inter > _IOU_THR * union),
                           1.0, 0.0).astype(jnp.bfloat16)


def _nms_max_body(cand_ref, mx_ref):
    mx_ref[...] = jnp.max(cand_ref[:, 2:6], keepdims=True)


def _nms_solve_body(m_ref, ct_ref, bxt_ref, sc_ref, lb_ref, *, N, NP):
    M = m_ref[...]

    def cond(carry):
        _, changed, it = carry
        return changed & (it < N + 2)

    def body(carry):
        alive, _, it = carry
        counts = jax.lax.dot_general(
            alive, M, (((1,), (0,)), ((), ())),
            preferred_element_type=jnp.float32)
        new_alive = (1.0 - jnp.minimum(counts[:, :N], 1.0)).astype(jnp.bfloat16)
        ndiff = jnp.sum(jnp.abs((new_alive - alive).astype(jnp.float32)))
        return new_alive, ndiff > 0.0, it + 1

    alive0 = jnp.ones((1, N), jnp.bfloat16)
    alive, _, _ = jax.lax.while_loop(
        cond, body, (alive0, jnp.bool_(True), jnp.int32(0)))
    keep = alive.astype(jnp.float32)
    bxt_ref[...] = ct_ref[2:6, 0:N] * keep
    sc_ref[...] = ct_ref[0:1, 0:N] * keep
    lb_ref[...] = ((ct_ref[1:2, 0:N] + 1.0) * keep).astype(jnp.int32)


def _run_nms(cand, cand_t, N, NP):
    CH = 256
    mx = pl.pallas_call(
        _nms_max_body,
        out_shape=jax.ShapeDtypeStruct((1, 1), jnp.float32),
    )(cand)
    M = pl.pallas_call(
        functools.partial(_nms_mat_body, N=N, CH=CH),
        grid=(NP // CH,),
        in_specs=[pl.BlockSpec((N, 8), lambda i: (0, 0)),
                  pl.BlockSpec((8, CH), lambda i: (0, i)),
                  pl.BlockSpec((1, 1), lambda i: (0, 0))],
        out_specs=pl.BlockSpec((N, CH), lambda i: (0, i)),
        out_shape=jax.ShapeDtypeStruct((N, NP), jnp.bfloat16),
    )(cand, cand_t, mx)
    return pl.pallas_call(
        functools.partial(_nms_solve_body, N=N, NP=NP),
        out_shape=(jax.ShapeDtypeStruct((4, N), jnp.float32),
                   jax.ShapeDtypeStruct((1, N), jnp.float32),
                   jax.ShapeDtypeStruct((1, N), jnp.int32)),
    )(M, cand_t)


def kernel(feat_p3, feat_p4, feat_p5, feat_p6, feat_p7,
           cls_params, box_params, image_sizes):
    del image_sizes
    feats = (feat_p3, feat_p4, feat_p5, feat_p6, feat_p7)
    packed = []
    for f, (H, W, s, r) in zip(feats, _LEVELS):
        HW = H * W
        cls_out = _head(f, cls_params)
        box_out = _head(f, box_params)
        cls_hw = jnp.transpose(cls_out, (0, 2, 3, 1)).reshape(HW, _CLASS_NUM)
        box_hw = jnp.transpose(box_out, (0, 2, 3, 1)).reshape(HW, 4)
        cand = _run_decode(cls_hw, box_hw, H, W, s, r)
        if HW > _NMS_PRE:
            cand = _run_topk(cand, cand.T, HW, _NMS_PRE)
        packed.append(cand)
    cand = jnp.concatenate(packed, axis=0)
    N = cand.shape[0]
    NP = 2560
    cand_t = jnp.concatenate(
        [cand.T, jnp.zeros((8, NP - N), jnp.float32)], axis=1)
    bxt, scores, labels = _run_nms(cand, cand_t, N, NP)
    return (bxt.T[None], scores, labels)
